# baseline (speedup 1.0000x reference)
import functools

import jax
import jax.numpy as jnp
from jax.experimental import pallas as pl
from jax.experimental.pallas import tpu as pltpu


def _conv_body(n_taps, c4, row_stride, l_out_p, oh, ow,
               w_ref, b_ref, x_ref, o_ref):
    xf = x_ref[0]
    lx = xf.shape[1]
    kb = int(round(n_taps ** 0.5))
    pieces = []
    for i in range(kb):
        for j in range(kb):
            off = i * row_stride + j
            if off == 0:
                xfo = xf
            else:
                xfo = pltpu.roll(xf, lx - off, axis=1)
            pieces.append(xfo[:, :l_out_p])
    s = jnp.concatenate(pieces, axis=0)
    acc = jnp.dot(w_ref[...], s, preferred_element_type=jnp.float32)
    acc = acc + b_ref[...]
    acc = jnp.maximum(acc, 0.0).astype(o_ref.dtype)
    for y in range(oh):
        o_ref[0, :, y, :] = acc[:, y * row_stride:y * row_stride + ow]


def _conv2d_bias_relu(x, weight, bias, stride):
    n, c_in, h, w = x.shape
    c_out, _, k, _ = weight.shape
    s = int(stride)
    out_dtype = x.dtype

    oh = (h - k) // s + 1
    ow = (w - k) // s + 1

    kb = -(-k // s)
    hp = -(-h // s) * s
    wp = -(-w // s) * s
    h2, w2 = hp // s, wp // s
    c4 = c_in * s * s

    row_stride = w2
    l_out_p = oh * row_stride
    lx = h2 * row_stride

    x_p = jnp.pad(x, ((0, 0), (0, 0), (0, hp - h), (0, wp - w)))
    x_s2d = x_p.reshape(n, c_in, h2, s, w2, s)
    x_s2d = jnp.transpose(x_s2d, (0, 1, 3, 5, 2, 4))
    x_kern = x_s2d.reshape(n, c4, lx)

    kpad = kb * s
    w_p = jnp.pad(weight, ((0, 0), (0, 0), (0, kpad - k), (0, kpad - k)))
    w_r = w_p.reshape(c_out, c_in, kb, s, kb, s)
    w_r = jnp.transpose(w_r, (0, 2, 4, 1, 3, 5))
    w_kern = w_r.reshape(c_out, kb * kb * c4).astype(out_dtype)

    b_kern = bias.reshape(c_out, 1).astype(jnp.float32)

    n_taps = kb * kb
    body = functools.partial(_conv_body, n_taps, c4, row_stride, l_out_p,
                             oh, ow)

    cost = pl.CostEstimate(
        flops=2 * n * c_out * n_taps * c4 * l_out_p,
        transcendentals=0,
        bytes_accessed=(x_kern.size + w_kern.size + b_kern.size
                        + n * c_out * oh * ow) * 4,
    )

    out = pl.pallas_call(
        body,
        out_shape=jax.ShapeDtypeStruct((n, c_out, oh, ow), out_dtype),
        grid=(n,),
        in_specs=[
            pl.BlockSpec((c_out, n_taps * c4), lambda b: (0, 0)),
            pl.BlockSpec((c_out, 1), lambda b: (0, 0)),
            pl.BlockSpec((1, c4, lx), lambda b: (b, 0, 0)),
        ],
        out_specs=pl.BlockSpec((1, c_out, oh, ow), lambda b: (b, 0, 0, 0)),
        compiler_params=pltpu.CompilerParams(
            dimension_semantics=("parallel",),
            vmem_limit_bytes=64 * 1024 * 1024,
        ),
        cost_estimate=cost,
    )(w_kern, b_kern, x_kern)

    return out


def kernel(x, weight, bias):
    return _conv2d_bias_relu(x, weight, bias, 2)


# speedup vs baseline: 2.2832x; 2.2832x over previous
import functools

import jax
import jax.numpy as jnp
from jax.experimental import pallas as pl
from jax.experimental.pallas import tpu as pltpu


def _round_up(x, m):
    return ((x + m - 1) // m) * m


def _conv_body(n_taps, c4p, row_stride, l_out_p, oh, ow,
               w_ref, b_ref, x_ref, o_ref):
    xf = x_ref[0]
    lx = xf.shape[1]
    kb = int(round(n_taps ** 0.5))
    pieces = []
    for j in range(kb):
        if j == 0:
            xfj = xf
        else:
            xfj = pltpu.roll(xf, lx - j, axis=1)
        for i in range(kb):
            base = i * row_stride
            pieces.append(xfj[:, base:base + l_out_p])
    s = jnp.concatenate(pieces, axis=0)
    acc = jnp.dot(w_ref[...], s, preferred_element_type=jnp.float32)
    acc = acc + b_ref[...]
    acc = jnp.maximum(acc, 0.0).astype(o_ref.dtype)
    for y in range(oh):
        o_ref[0, :, y, :] = acc[:, y * row_stride:y * row_stride + ow]


def _conv2d_bias_relu(x, weight, bias, stride):
    n, c_in, h, w = x.shape
    c_out, _, k, _ = weight.shape
    s = int(stride)
    out_dtype = x.dtype

    oh = (h - k) // s + 1
    ow = (w - k) // s + 1

    kb = -(-k // s)
    hp = -(-h // s) * s
    c4 = c_in * s * s
    c4p = _round_up(c4, 8)
    c_in_p = c4p // (s * s)

    w2 = hp // s
    row_stride = _round_up(w2 + kb - 1, 128)
    lx = (hp // s) * row_stride
    l_out_p = oh * row_stride

    x_p = jnp.pad(
        x, ((0, 0), (0, c_in_p - c_in), (0, hp - h), (0, s * row_stride - w)))
    x_p = x_p.reshape(n, c_in_p, hp // s, s, row_stride, s)
    x_s2d = jnp.transpose(x_p, (0, 1, 3, 5, 2, 4))
    x_kern = x_s2d.reshape(n, c4p, lx)

    kpad = kb * s
    w_p = jnp.pad(weight, ((0, 0), (0, c_in_p - c_in),
                           (0, kpad - k), (0, kpad - k)))
    w_r = w_p.reshape(c_out, c_in_p, kb, s, kb, s)
    w_r = jnp.transpose(w_r, (0, 4, 2, 1, 3, 5))
    w_kern = w_r.reshape(c_out, kb * kb * c4p).astype(out_dtype)

    b_kern = bias.reshape(c_out, 1).astype(jnp.float32)

    n_taps = kb * kb
    body = functools.partial(_conv_body, n_taps, c4p, row_stride, l_out_p,
                             oh, ow)

    cost = pl.CostEstimate(
        flops=2 * n * c_out * n_taps * c4p * l_out_p,
        transcendentals=0,
        bytes_accessed=(x_kern.size + w_kern.size + b_kern.size
                        + n * c_out * oh * ow) * 4,
    )

    out = pl.pallas_call(
        body,
        out_shape=jax.ShapeDtypeStruct((n, c_out, oh, ow), out_dtype),
        grid=(n,),
        in_specs=[
            pl.BlockSpec((c_out, n_taps * c4p), lambda b: (0, 0)),
            pl.BlockSpec((c_out, 1), lambda b: (0, 0)),
            pl.BlockSpec((1, c4p, lx), lambda b: (b, 0, 0)),
        ],
        out_specs=pl.BlockSpec((1, c_out, oh, ow), lambda b: (b, 0, 0, 0)),
        compiler_params=pltpu.CompilerParams(
            dimension_semantics=("parallel",),
            vmem_limit_bytes=64 * 1024 * 1024,
        ),
        cost_estimate=cost,
    )(w_kern, b_kern, x_kern)

    return out


def kernel(x, weight, bias):
    return _conv2d_bias_relu(x, weight, bias, 2)


# speedup vs baseline: 2.4318x; 1.0651x over previous
import functools

import jax
import jax.numpy as jnp
from jax.experimental import pallas as pl
from jax.experimental.pallas import tpu as pltpu


def _round_up(x, m):
    return ((x + m - 1) // m) * m


def _conv_body(n_taps, c4p, row_stride, l_out_p, oh, ow,
               w_ref, b_ref, x_ref, o_ref):
    xf = x_ref[0]
    lx = xf.shape[1]
    kb = int(round(n_taps ** 0.5))
    pieces = []
    for j in range(kb):
        if j == 0:
            xfj = xf
        else:
            xfj = pltpu.roll(xf, lx - j, axis=1)
        for i in range(kb):
            base = i * row_stride
            pieces.append(xfj[:, base:base + l_out_p])
    s = jnp.concatenate(pieces, axis=0)
    acc = jnp.dot(w_ref[...], s, preferred_element_type=jnp.float32)
    acc = acc + b_ref[...]
    acc = jnp.maximum(acc, 0.0).astype(o_ref.dtype)
    for y in range(oh):
        o_ref[0, :, y, :] = acc[:, y * row_stride:y * row_stride + ow]


def _conv2d_bias_relu(x, weight, bias, stride):
    n, c_in, h, w = x.shape
    c_out, _, k, _ = weight.shape
    s = int(stride)
    out_dtype = x.dtype

    oh = (h - k) // s + 1
    ow = (w - k) // s + 1

    kb = -(-k // s)
    hp = -(-h // s) * s
    wp = -(-w // s) * s
    h2, w2 = hp // s, wp // s
    c4 = c_in * s * s
    c4p = _round_up(c4, 8)

    row_stride = _round_up(w2 + kb - 1, 128)
    lx = h2 * row_stride
    l_out_p = oh * row_stride

    x_p = jnp.pad(x, ((0, 0), (0, 0), (0, hp - h), (0, wp - w)))
    x_s2d = x_p.reshape(n, c_in, h2, s, w2, s)
    x_s2d = jnp.transpose(x_s2d, (0, 1, 3, 5, 2, 4))
    x_s2d = x_s2d.reshape(n, c4, h2, w2)
    x_s2d = jnp.pad(
        x_s2d, ((0, 0), (0, c4p - c4), (0, 0), (0, row_stride - w2)))
    x_kern = x_s2d.reshape(n, c4p, lx)

    kpad = kb * s
    w_p = jnp.pad(weight, ((0, 0), (0, 0), (0, kpad - k), (0, kpad - k)))
    w_r = w_p.reshape(c_out, c_in, kb, s, kb, s)
    w_r = jnp.transpose(w_r, (0, 4, 2, 1, 3, 5))
    w_r = w_r.reshape(c_out, kb * kb, c4)
    w_r = jnp.pad(w_r, ((0, 0), (0, 0), (0, c4p - c4)))
    w_kern = w_r.reshape(c_out, kb * kb * c4p).astype(out_dtype)

    b_kern = bias.reshape(c_out, 1).astype(jnp.float32)

    n_taps = kb * kb
    body = functools.partial(_conv_body, n_taps, c4p, row_stride, l_out_p,
                             oh, ow)

    cost = pl.CostEstimate(
        flops=2 * n * c_out * n_taps * c4p * l_out_p,
        transcendentals=0,
        bytes_accessed=(x_kern.size + w_kern.size + b_kern.size
                        + n * c_out * oh * ow) * 4,
    )

    out = pl.pallas_call(
        body,
        out_shape=jax.ShapeDtypeStruct((n, c_out, oh, ow), out_dtype),
        grid=(n,),
        in_specs=[
            pl.BlockSpec((c_out, n_taps * c4p), lambda b: (0, 0)),
            pl.BlockSpec((c_out, 1), lambda b: (0, 0)),
            pl.BlockSpec((1, c4p, lx), lambda b: (b, 0, 0)),
        ],
        out_specs=pl.BlockSpec((1, c_out, oh, ow), lambda b: (b, 0, 0, 0)),
        compiler_params=pltpu.CompilerParams(
            dimension_semantics=("parallel",),
            vmem_limit_bytes=64 * 1024 * 1024,
        ),
        cost_estimate=cost,
    )(w_kern, b_kern, x_kern)

    return out


def kernel(x, weight, bias):
    return _conv2d_bias_relu(x, weight, bias, 2)
